# baseline (speedup 1.0000x reference)
import functools

import jax
import jax.numpy as jnp
from jax import lax
from jax.experimental import pallas as pl
from jax.experimental.pallas import tpu as pltpu
from jax.experimental.pallas import tpu_sc as plsc

N2 = 16
NF = 7
W = 128
K = 8
LANES = 16


def _sc_gather(tabs, idx3, b, l):
    mesh = plsc.VectorSubcoreMesh(core_axis_name="c", subcore_axis_name="s")
    out = jax.ShapeDtypeStruct((l, NF * N2 // 8, b // W, 8, W), jnp.float32)
    bblocks = b // (K * W)

    @functools.partial(
        pl.kernel, out_type=(out, out), mesh=mesh,
        scratch_types=[pltpu.VMEM((K * W, N2), jnp.float32),
                       pltpu.VMEM((K * W, N2), jnp.float32),
                       pltpu.SemaphoreType.DMA],
        compiler_params=pltpu.CompilerParams(use_tc_tiling_on_sc=False,
                                             needs_layout_passes=False))
    def run(*refs):
        tab_refs = refs[:2 * NF]
        idx_hbm, ore_hbm, oim_hbm, scr_re, scr_im, sem = refs[2 * NF:]
        re_refs = tab_refs[0::2]
        im_refs = tab_refs[1::2]

        def body(idxs, idx_v, ore_v, oim_v):
            _, f = idxs
            for ff in range(NF):
                @pl.when(f == ff)
                def _(ff=ff):
                    for k in range(K):
                        rows_ref = idx_v.at[0, k]
                        dst = pl.ds(k * W, W)
                        pltpu.async_copy(re_refs[ff].at[rows_ref],
                                         scr_re.at[dst, :], sem)
                        pltpu.async_copy(im_refs[ff].at[rows_ref],
                                         scr_im.at[dst, :], sem)
            for k in range(K):
                rows_ref = idx_v.at[0, k]
                dst = pl.ds(k * W, W)
                pltpu.make_async_copy(re_refs[0].at[rows_ref],
                                      scr_re.at[dst, :], sem).wait()
                pltpu.make_async_copy(im_refs[0].at[rows_ref],
                                      scr_im.at[dst, :], sem).wait()

            lanes = lax.iota(jnp.int32, LANES)

            @pl.loop(0, K)
            def _(k):
                @pl.loop(0, N2)
                def _(c):
                    ch = c // 8
                    cl = c % 8
                    cols = jnp.full((LANES,), c, jnp.int32)
                    for b16 in range(W // LANES):
                        rows = lanes + (k * W + b16 * LANES)
                        dst = pl.ds(b16 * LANES, LANES)
                        ore_v[0, ch, k, cl, dst] = plsc.load_gather(
                            scr_re, [rows, cols])
                        oim_v[0, ch, k, cl, dst] = plsc.load_gather(
                            scr_im, [rows, cols])

        out_spec = pl.BlockSpec(
            (1, 2, K, 8, W),
            lambda i, f: (i // bblocks, f, i % bblocks, 0, 0))
        pltpu.emit_pipeline(
            body,
            grid=(b * l // (K * W), NF),
            in_specs=[pl.BlockSpec((1, K, W), lambda i, f: (f, i, 0))],
            out_specs=[out_spec, out_spec],
            core_axis_name=("c", "s"),
            dimension_semantics=(pltpu.PARALLEL, pltpu.PARALLEL),
            _explicit_indices=True,
        )(idx_hbm, ore_hbm, oim_hbm)

    return run(*tabs, idx3)


def kernel(data, yr_re, yr_im, mt_re, mt_im, x_re, x_im, y_re, y_im,
           m_re, m_im, d_re, d_im, t_re, t_im):
    b, l, _ = data.shape
    tabs = (yr_re, yr_im, mt_re, mt_im, x_re, x_im, y_re, y_im,
            m_re, m_im, d_re, d_im, t_re, t_im)
    idx3 = jnp.transpose(data, (2, 1, 0)).reshape(NF, b * l // W, W)
    ore5, oim5 = _sc_gather(tabs, idx3, b, l)

    def unview(o5):
        return o5.transpose(2, 4, 0, 1, 3).reshape(b, l, NF * N2)

    return unview(ore5), unview(oim5)


# speedup vs baseline: 1.0586x; 1.0586x over previous
import functools

import jax
import jax.numpy as jnp
from jax import lax
from jax.experimental import pallas as pl
from jax.experimental.pallas import tpu as pltpu
from jax.experimental.pallas import tpu_sc as plsc

N2 = 16
NF = 7
W = 128
K = 8
LANES = 16


def _sc_gather(tabs, idx3, b, l):
    mesh = plsc.VectorSubcoreMesh(core_axis_name="c", subcore_axis_name="s")
    out = jax.ShapeDtypeStruct((l, NF * N2 // 8, b // W, 8, W), jnp.float32)
    bblocks = b // (K * W)

    @functools.partial(
        pl.kernel, out_type=(out, out), mesh=mesh,
        scratch_types=[pltpu.VMEM((K * W, N2), jnp.float32),
                       pltpu.VMEM((K * W, N2), jnp.float32),
                       pltpu.SemaphoreType.DMA],
        compiler_params=pltpu.CompilerParams(use_tc_tiling_on_sc=False,
                                             needs_layout_passes=False))
    def run(*refs):
        tab_refs = refs[:2 * NF]
        idx_hbm, ore_hbm, oim_hbm, scr_re, scr_im, sem = refs[2 * NF:]
        re_refs = tab_refs[0::2]
        im_refs = tab_refs[1::2]

        def body(idxs, idx_v, ore_v, oim_v):
            _, f = idxs
            for ff in range(NF):
                @pl.when(f == ff)
                def _(ff=ff):
                    for k in range(K):
                        rows_ref = idx_v.at[0, k]
                        dst = pl.ds(k * W, W)
                        pltpu.async_copy(re_refs[ff].at[rows_ref],
                                         scr_re.at[dst, :], sem)
                        pltpu.async_copy(im_refs[ff].at[rows_ref],
                                         scr_im.at[dst, :], sem)
            lanes = lax.iota(jnp.int32, LANES)
            for k in range(K):
                rows_ref = idx_v.at[0, k]
                dst = pl.ds(k * W, W)
                pltpu.make_async_copy(re_refs[0].at[rows_ref],
                                      scr_re.at[dst, :], sem).wait()
                pltpu.make_async_copy(im_refs[0].at[rows_ref],
                                      scr_im.at[dst, :], sem).wait()
                for c in range(N2):
                    cols = jnp.full((LANES,), c, jnp.int32)
                    for b16 in range(W // LANES):
                        rows = lanes + (k * W + b16 * LANES)
                        dstb = pl.ds(b16 * LANES, LANES)
                        ore_v[0, c // 8, k, c % 8, dstb] = plsc.load_gather(
                            scr_re, [rows, cols])
                        oim_v[0, c // 8, k, c % 8, dstb] = plsc.load_gather(
                            scr_im, [rows, cols])

        out_spec = pl.BlockSpec(
            (1, 2, K, 8, W),
            lambda i, f: (i // bblocks, f, i % bblocks, 0, 0))
        pltpu.emit_pipeline(
            body,
            grid=(b * l // (K * W), NF),
            in_specs=[pl.BlockSpec((1, K, W), lambda i, f: (f, i, 0))],
            out_specs=[out_spec, out_spec],
            core_axis_name=("c", "s"),
            dimension_semantics=(pltpu.PARALLEL, pltpu.PARALLEL),
            _explicit_indices=True,
        )(idx_hbm, ore_hbm, oim_hbm)

    return run(*tabs, idx3)


def kernel(data, yr_re, yr_im, mt_re, mt_im, x_re, x_im, y_re, y_im,
           m_re, m_im, d_re, d_im, t_re, t_im):
    b, l, _ = data.shape
    tabs = (yr_re, yr_im, mt_re, mt_im, x_re, x_im, y_re, y_im,
            m_re, m_im, d_re, d_im, t_re, t_im)
    idx3 = jnp.transpose(data, (2, 1, 0)).reshape(NF, b * l // W, W)
    ore5, oim5 = _sc_gather(tabs, idx3, b, l)

    def unview(o5):
        return o5.transpose(2, 4, 0, 1, 3).reshape(b, l, NF * N2)

    return unview(ore5), unview(oim5)
